# baseline (speedup 1.0000x reference)
import functools

import jax
import jax.numpy as jnp
from jax.experimental import pallas as pl
from jax.experimental.pallas import tpu as pltpu

_CBLK = 192


def _loss_kernel(en_ref, de_ref, out_ref, acc_ref, *, nc):
    b = pl.program_id(0)
    j = pl.program_id(1)

    @pl.when(j == 0)
    def _init():
        acc_ref[0] = 0.0
        acc_ref[1] = 0.0
        acc_ref[2] = 0.0

    en = en_ref[0]
    de = de_ref[0]

    acc_ref[0] += jnp.sum(en * de)
    acc_ref[1] += jnp.sum(en * en)
    acc_ref[2] += jnp.sum(de * de)

    @pl.when(j == nc - 1)
    def _finalize():
        dot, na2, nb2 = acc_ref[0], acc_ref[1], acc_ref[2]
        term = 1.0 - dot / jnp.maximum(jnp.sqrt(na2) * jnp.sqrt(nb2), 1e-8)

        @pl.when(b == 0)
        def _first():
            out_ref[0, 0] = term

        @pl.when(b > 0)
        def _rest():
            out_ref[0, 0] += term


def kernel(encoder_features, decoder_features, global_step):
    B, C, H, W = encoder_features.shape
    nc = C // _CBLK

    out = pl.pallas_call(
        functools.partial(_loss_kernel, nc=nc),
        grid=(B, nc),
        in_specs=[
            pl.BlockSpec((1, _CBLK, H, W), lambda b, j: (b, j, 0, 0)),
            pl.BlockSpec((1, _CBLK, H, W), lambda b, j: (b, j, 0, 0)),
        ],
        out_specs=pl.BlockSpec(
            (1, 1), lambda b, j: (0, 0), memory_space=pltpu.SMEM
        ),
        out_shape=jax.ShapeDtypeStruct((1, 1), jnp.float32),
        scratch_shapes=[pltpu.SMEM((3,), jnp.float32)],
    )(encoder_features, decoder_features)
    return (out[0, 0] / B).reshape(())


# speedup vs baseline: 12.1716x; 12.1716x over previous
import jax
import jax.numpy as jnp
from jax.experimental import pallas as pl
from jax.experimental.pallas import tpu as pltpu


def _loss_kernel(en_ref, de_ref, out_ref):
    b = pl.program_id(0)
    en = en_ref[0]
    de = de_ref[0]

    dot = jnp.sum(en * de)
    na2 = jnp.sum(en * en)
    nb2 = jnp.sum(de * de)
    term = 1.0 - dot / jnp.maximum(jnp.sqrt(na2) * jnp.sqrt(nb2), 1e-8)

    @pl.when(b == 0)
    def _first():
        out_ref[0, 0] = term

    @pl.when(b > 0)
    def _rest():
        out_ref[0, 0] += term


def kernel(encoder_features, decoder_features, global_step):
    B, C, H, W = encoder_features.shape
    en = jnp.transpose(encoder_features, (0, 2, 3, 1))
    de = jnp.transpose(decoder_features, (0, 2, 3, 1))

    out = pl.pallas_call(
        _loss_kernel,
        grid=(B,),
        in_specs=[
            pl.BlockSpec((1, H, W, C), lambda b: (b, 0, 0, 0)),
            pl.BlockSpec((1, H, W, C), lambda b: (b, 0, 0, 0)),
        ],
        out_specs=pl.BlockSpec(
            (1, 1), lambda b: (0, 0), memory_space=pltpu.SMEM
        ),
        out_shape=jax.ShapeDtypeStruct((1, 1), jnp.float32),
    )(en, de)
    return (out[0, 0] / B).reshape(())
